# baseline (speedup 1.0000x reference)
import functools

import jax
import jax.numpy as jnp
from jax import lax
from jax.experimental import pallas as pl
from jax.experimental.pallas import tpu as pltpu
from jax.experimental.pallas import tpu_sc as plsc

_PARTS = [[0, 1, 2, 3, 20], [4, 5, 6, 7, 21, 22], [8, 9, 10, 11, 23, 24],
          [12, 13, 14, 15], [16, 17, 18, 19]]
_V_OUT = 25
_NC, _NS = 2, 16
_NW = _NC * _NS
_N, _C, _T = 64, 256, 128
_NB = 3

_mesh = plsc.VectorSubcoreMesh(core_axis_name="c", subcore_axis_name="s")


@functools.partial(
    pl.kernel,
    mesh=_mesh,
    out_type=jax.ShapeDtypeStruct((_N, _V_OUT, _C, _T), jnp.float32),
    scratch_types=[
        pltpu.VMEM((_NB, _C, _T), jnp.float32),
        pltpu.SemaphoreType.DMA,
        pltpu.SemaphoreType.DMA((_NB,)),
    ],
)
def _sc_copy(x_hbm, o_hbm, bufs, in_sem, out_sems):
    wid = lax.axis_index("s") * _NC + lax.axis_index("c")
    n_per_w = _N // _NW
    pending = [[] for _ in range(_NB)]
    step = 0
    for k in range(n_per_w):
        n = wid * n_per_w + k
        for i, part in enumerate(_PARTS):
            b = step % _NB
            for cp in pending[b]:
                cp.wait()
            pending[b] = []
            pltpu.async_copy(x_hbm.at[n, i], bufs.at[b], in_sem).wait()
            for v in part:
                cp = pltpu.async_copy(bufs.at[b], o_hbm.at[n, v],
                                      out_sems.at[b])
                pending[b].append(cp)
            step += 1
    for b in range(_NB):
        for cp in pending[b]:
            cp.wait()


def kernel(x):
    N, C, T, V = x.shape
    xt = jnp.transpose(x, (0, 3, 1, 2))
    out_t = _sc_copy(xt)
    return jnp.transpose(out_t, (0, 2, 3, 1))


# speedup vs baseline: 1.0170x; 1.0170x over previous
import functools

import jax
import jax.numpy as jnp
from jax import lax
from jax.experimental import pallas as pl
from jax.experimental.pallas import tpu as pltpu
from jax.experimental.pallas import tpu_sc as plsc

_PARTS = [[0, 1, 2, 3, 20], [4, 5, 6, 7, 21, 22], [8, 9, 10, 11, 23, 24],
          [12, 13, 14, 15], [16, 17, 18, 19]]
_V_OUT = 25
_NC, _NS = 2, 16
_NW = _NC * _NS
_N, _C, _T = 64, 256, 128
_NB = 3

_mesh = plsc.VectorSubcoreMesh(core_axis_name="c", subcore_axis_name="s")


@functools.partial(
    pl.kernel,
    mesh=_mesh,
    out_type=jax.ShapeDtypeStruct((_N, _V_OUT, _C, _T), jnp.float32),
    scratch_types=[
        pltpu.VMEM((_NB, _C, _T), jnp.float32),
        pltpu.SemaphoreType.DMA((_NB,)),
        pltpu.SemaphoreType.DMA((_NB,)),
    ],
)
def _sc_copy(x_hbm, o_hbm, bufs, in_sems, out_sems):
    wid = lax.axis_index("s") * _NC + lax.axis_index("c")
    n_per_w = _N // _NW
    steps = [(wid * n_per_w + k, i, part)
             for k in range(n_per_w) for i, part in enumerate(_PARTS)]
    pending = [[] for _ in range(_NB)]
    in_cps = {}

    def start_in(s):
        n, i, _ = steps[s]
        b = s % _NB
        for cp in pending[b]:
            cp.wait()
        pending[b] = []
        in_cps[s] = pltpu.async_copy(x_hbm.at[n, i], bufs.at[b],
                                     in_sems.at[b])

    start_in(0)
    start_in(1)
    for s, (n, i, part) in enumerate(steps):
        b = s % _NB
        in_cps.pop(s).wait()
        for v in part:
            cp = pltpu.async_copy(bufs.at[b], o_hbm.at[n, v], out_sems.at[b])
            pending[b].append(cp)
        if s + 2 < len(steps):
            start_in(s + 2)
    for b in range(_NB):
        for cp in pending[b]:
            cp.wait()


def kernel(x):
    N, C, T, V = x.shape
    xt = jnp.transpose(x, (0, 3, 1, 2))
    out_t = _sc_copy(xt)
    return jnp.transpose(out_t, (0, 2, 3, 1))
